# Initial kernel scaffold; baseline (speedup 1.0000x reference)
import jax
import jax.numpy as jnp
from jax.experimental import pallas as pl


def kernel(edge_index, edge_weight, x, W1, b1, W2, b2):
    raise NotImplementedError("write your pallas kernel here")


# speedup vs baseline: 10.2302x; 10.2302x over previous
import functools

import jax
import jax.numpy as jnp
from jax import lax
from jax.experimental import pallas as pl
from jax.experimental.pallas import tpu as pltpu
from jax.experimental.pallas import tpu_sc as plsc

N_NODES = 10000
N_PAD = 10240
N_EDGES = 320000
IN_DIM = 128
HID = 32
OUT = 7
OUT_PAD = 16

NC = 2
NS = 16
LANES = 16
NW = NC * NS
EPW = N_EDGES // NW
GE = 80
NG = EPW // GE


def _make_sc_propagate(d):
    rows_per_sub = N_PAD // NS

    mesh = plsc.VectorSubcoreMesh(core_axis_name="c", subcore_axis_name="s")

    @functools.partial(
        pl.kernel,
        out_type=jax.ShapeDtypeStruct((NC, N_PAD, d), jnp.float32),
        mesh=mesh,
        compiler_params=pltpu.CompilerParams(use_tc_tiling_on_sc=False),
        scratch_types=[
            pltpu.VMEM((NG, GE), jnp.int32),
            pltpu.VMEM((NG, GE), jnp.int32),
            pltpu.VMEM((NG, GE), jnp.float32),
            pltpu.VMEM((GE, d), jnp.float32),
            pltpu.VMEM_SHARED((N_PAD, d), jnp.float32),
            pltpu.SemaphoreType.DMA,
        ],
    )
    def propagate(src_hbm, dst_hbm, w_hbm, y_hbm, z_hbm, out_hbm,
                  src_v, dst_v, w_v, rows_v, acc, sem):
        c = lax.axis_index("c")
        s = lax.axis_index("s")
        wid = c * NS + s

        pltpu.sync_copy(src_hbm.at[wid], src_v)
        pltpu.sync_copy(dst_hbm.at[wid], dst_v)
        pltpu.sync_copy(w_hbm.at[wid], w_v)

        pltpu.sync_copy(z_hbm.at[pl.ds(s * rows_per_sub, rows_per_sub)],
                        acc.at[pl.ds(s * rows_per_sub, rows_per_sub)])
        plsc.subcore_barrier()

        def group(g, carry):
            pltpu.async_copy(y_hbm.at[src_v.at[g]], rows_v, sem).wait()
            for e16 in range(GE // LANES):
                wv = w_v[g, pl.ds(e16 * LANES, LANES)]
                for j in range(LANES):
                    e = e16 * LANES + j
                    we = wv[j]
                    for f0 in range(0, d, LANES):
                        rows_v[e, pl.ds(f0, LANES)] = (
                            rows_v[e, pl.ds(f0, LANES)] * we)
            pltpu.sync_copy(rows_v, acc.at[dst_v.at[g]], add=True)
            return carry

        lax.fori_loop(0, NG, group, 0)
        plsc.subcore_barrier()

        pltpu.sync_copy(acc.at[pl.ds(s * rows_per_sub, rows_per_sub)],
                        out_hbm.at[c, pl.ds(s * rows_per_sub, rows_per_sub)])

    return propagate


_sc_prop_hid = _make_sc_propagate(HID)
_sc_prop_out = _make_sc_propagate(OUT_PAD)


def _mm1_body(x_ref, w_ref, o_ref):
    o_ref[...] = jnp.dot(x_ref[...], w_ref[...],
                         preferred_element_type=jnp.float32)


def _mm1(x, w1):
    return pl.pallas_call(
        _mm1_body,
        grid=(10,),
        in_specs=[
            pl.BlockSpec((N_NODES // 10, IN_DIM), lambda i: (i, 0)),
            pl.BlockSpec((IN_DIM, HID), lambda i: (0, 0)),
        ],
        out_specs=pl.BlockSpec((N_NODES // 10, HID), lambda i: (i, 0)),
        out_shape=jax.ShapeDtypeStruct((N_NODES, HID), jnp.float32),
    )(x, w1)


def _mid_body(pa_ref, pb_ref, b1_ref, w2_ref, o_ref):
    h = jnp.maximum(pa_ref[...] + pb_ref[...] + b1_ref[...], 0.0)
    o_ref[...] = jnp.dot(h, w2_ref[...], preferred_element_type=jnp.float32)


def _mid(pa, pb, b1, w2p):
    blk = N_PAD // 10
    return pl.pallas_call(
        _mid_body,
        grid=(10,),
        in_specs=[
            pl.BlockSpec((blk, HID), lambda i: (i, 0)),
            pl.BlockSpec((blk, HID), lambda i: (i, 0)),
            pl.BlockSpec((1, HID), lambda i: (0, 0)),
            pl.BlockSpec((HID, OUT_PAD), lambda i: (0, 0)),
        ],
        out_specs=pl.BlockSpec((blk, OUT_PAD), lambda i: (i, 0)),
        out_shape=jax.ShapeDtypeStruct((N_PAD, OUT_PAD), jnp.float32),
    )(pa, pb, b1, w2p)


def _fin_body(pa_ref, pb_ref, b2_ref, o_ref):
    o_ref[...] = pa_ref[...] + pb_ref[...] + b2_ref[...]


def _fin(pa, pb, b2p):
    blk = N_PAD // 10
    return pl.pallas_call(
        _fin_body,
        grid=(10,),
        in_specs=[
            pl.BlockSpec((blk, OUT_PAD), lambda i: (i, 0)),
            pl.BlockSpec((blk, OUT_PAD), lambda i: (i, 0)),
            pl.BlockSpec((1, OUT_PAD), lambda i: (0, 0)),
        ],
        out_specs=pl.BlockSpec((blk, OUT_PAD), lambda i: (i, 0)),
        out_shape=jax.ShapeDtypeStruct((N_PAD, OUT_PAD), jnp.float32),
    )(pa, pb, b2p)


@jax.jit
def _run(edge_index, edge_weight, x, w1, b1, w2, b2):
    src = edge_index[0].astype(jnp.int32).reshape(NW, NG, GE)
    dst = edge_index[1].astype(jnp.int32).reshape(NW, NG, GE)
    w3 = edge_weight.reshape(NW, NG, GE)

    z_hid = jnp.zeros((N_PAD, HID), jnp.float32)
    z_out = jnp.zeros((N_PAD, OUT_PAD), jnp.float32)
    w2p = jnp.pad(w2, ((0, 0), (0, OUT_PAD - OUT)))
    b2p = jnp.pad(b2, (0, OUT_PAD - OUT)).reshape(1, OUT_PAD)
    b1r = b1.reshape(1, HID)

    y1 = _mm1(x, w1)
    p1 = _sc_prop_hid(src, dst, w3, y1, z_hid)
    y2 = _mid(p1[0], p1[1], b1r, w2p)
    p2 = _sc_prop_out(src, dst, w3, y2[:N_NODES], z_out)
    out = _fin(p2[0], p2[1], b2p)
    return out[:N_NODES, :OUT]


def kernel(edge_index, edge_weight, x, W1, b1, W2, b2):
    return _run(edge_index, edge_weight, x, W1, b1, W2, b2)


# speedup vs baseline: 15.8836x; 1.5526x over previous
import functools

import jax
import jax.numpy as jnp
from jax import lax
from jax.experimental import pallas as pl
from jax.experimental.pallas import tpu as pltpu
from jax.experimental.pallas import tpu_sc as plsc

N_NODES = 10000
N_PAD = 10240
N_EDGES = 320000
IN_DIM = 128
HID = 32
OUT = 7
OUT_PAD = 16

NC = 2
NS = 16
LANES = 16
NW = NC * NS
EPW = N_EDGES // NW
GE = 80
NG = EPW // GE


def _make_sc_propagate(d):
    rows_per_sub = N_PAD // NS

    mesh = plsc.VectorSubcoreMesh(core_axis_name="c", subcore_axis_name="s")

    @functools.partial(
        pl.kernel,
        out_type=jax.ShapeDtypeStruct((NC, N_PAD, d), jnp.float32),
        mesh=mesh,
        compiler_params=pltpu.CompilerParams(use_tc_tiling_on_sc=False),
        scratch_types=[
            pltpu.VMEM((NG, GE), jnp.int32),
            pltpu.VMEM((NG, GE), jnp.int32),
            pltpu.VMEM((NG, GE), jnp.float32),
            pltpu.VMEM((GE, d), jnp.float32),
            pltpu.VMEM((GE, d), jnp.float32),
            pltpu.VMEM((GE, d), jnp.float32),
            pltpu.VMEM((GE, d), jnp.float32),
            pltpu.VMEM_SHARED((N_PAD, d), jnp.float32),
            pltpu.SemaphoreType.DMA,
            pltpu.SemaphoreType.DMA,
            pltpu.SemaphoreType.DMA,
            pltpu.SemaphoreType.DMA,
        ],
    )
    def propagate(src_hbm, dst_hbm, w_hbm, y_hbm, z_hbm, out_hbm,
                  src_v, dst_v, w_v, gbuf0, gbuf1, sbuf0, sbuf1, acc,
                  gsem0, gsem1, ssem0, ssem1):
        c = lax.axis_index("c")
        s = lax.axis_index("s")
        wid = c * NS + s
        gbufs = (gbuf0, gbuf1)
        sbufs = (sbuf0, sbuf1)
        gsems = (gsem0, gsem1)
        ssems = (ssem0, ssem1)

        pltpu.sync_copy(src_hbm.at[wid], src_v)
        pltpu.sync_copy(dst_hbm.at[wid], dst_v)
        pltpu.sync_copy(w_hbm.at[wid], w_v)

        pltpu.sync_copy(z_hbm.at[pl.ds(s * rows_per_sub, rows_per_sub)],
                        acc.at[pl.ds(s * rows_per_sub, rows_per_sub)])
        plsc.subcore_barrier()

        for b in range(2):
            pltpu.async_copy(y_hbm.at[src_v.at[b]], gbufs[b], gsems[b])

        def process(g, b):
            pltpu.make_async_copy(
                y_hbm.at[src_v.at[g]], gbufs[b], gsems[b]).wait()

            @pl.when(g >= 2)
            def _wait_prev_scatter():
                pltpu.make_async_copy(
                    sbufs[b], acc.at[dst_v.at[g]], ssems[b]).wait()

            for e16 in range(GE // LANES):
                wv = w_v[g, pl.ds(e16 * LANES, LANES)]
                for j in range(LANES):
                    e = e16 * LANES + j
                    we = wv[j]
                    for f0 in range(0, d, LANES):
                        sbufs[b][e, pl.ds(f0, LANES)] = (
                            gbufs[b][e, pl.ds(f0, LANES)] * we)

            pltpu.async_copy(
                sbufs[b], acc.at[dst_v.at[g]], ssems[b], add=True)

        def outer(i, carry):
            for b in range(2):
                g = i * 2 + b
                process(g, b)

                @pl.when(g + 2 < NG)
                def _issue_next_gather():
                    pltpu.async_copy(
                        y_hbm.at[src_v.at[g + 2]], gbufs[b], gsems[b])
            return carry

        lax.fori_loop(0, NG // 2, outer, 0)
        if NG % 2:
            process(NG - 1, (NG - 1) % 2)
        for b in range(2):
            pltpu.make_async_copy(
                sbufs[b], acc.at[dst_v.at[b]], ssems[b]).wait()
        plsc.subcore_barrier()

        pltpu.sync_copy(acc.at[pl.ds(s * rows_per_sub, rows_per_sub)],
                        out_hbm.at[c, pl.ds(s * rows_per_sub, rows_per_sub)])

    return propagate


_sc_prop_hid = _make_sc_propagate(HID)
_sc_prop_out = _make_sc_propagate(OUT_PAD)


def _mm1_body(x_ref, w_ref, o_ref):
    o_ref[...] = jnp.dot(x_ref[...], w_ref[...],
                         preferred_element_type=jnp.float32)


def _mm1(x, w1):
    return pl.pallas_call(
        _mm1_body,
        grid=(10,),
        in_specs=[
            pl.BlockSpec((N_NODES // 10, IN_DIM), lambda i: (i, 0)),
            pl.BlockSpec((IN_DIM, HID), lambda i: (0, 0)),
        ],
        out_specs=pl.BlockSpec((N_NODES // 10, HID), lambda i: (i, 0)),
        out_shape=jax.ShapeDtypeStruct((N_NODES, HID), jnp.float32),
    )(x, w1)


def _mid_body(pa_ref, pb_ref, b1_ref, w2_ref, o_ref):
    h = jnp.maximum(pa_ref[...] + pb_ref[...] + b1_ref[...], 0.0)
    o_ref[...] = jnp.dot(h, w2_ref[...], preferred_element_type=jnp.float32)


def _mid(pa, pb, b1, w2p):
    blk = N_PAD // 10
    return pl.pallas_call(
        _mid_body,
        grid=(10,),
        in_specs=[
            pl.BlockSpec((blk, HID), lambda i: (i, 0)),
            pl.BlockSpec((blk, HID), lambda i: (i, 0)),
            pl.BlockSpec((1, HID), lambda i: (0, 0)),
            pl.BlockSpec((HID, OUT_PAD), lambda i: (0, 0)),
        ],
        out_specs=pl.BlockSpec((blk, OUT_PAD), lambda i: (i, 0)),
        out_shape=jax.ShapeDtypeStruct((N_PAD, OUT_PAD), jnp.float32),
    )(pa, pb, b1, w2p)


def _fin_body(pa_ref, pb_ref, b2_ref, o_ref):
    o_ref[...] = pa_ref[...] + pb_ref[...] + b2_ref[...]


def _fin(pa, pb, b2p):
    blk = N_PAD // 10
    return pl.pallas_call(
        _fin_body,
        grid=(10,),
        in_specs=[
            pl.BlockSpec((blk, OUT_PAD), lambda i: (i, 0)),
            pl.BlockSpec((blk, OUT_PAD), lambda i: (i, 0)),
            pl.BlockSpec((1, OUT_PAD), lambda i: (0, 0)),
        ],
        out_specs=pl.BlockSpec((blk, OUT_PAD), lambda i: (i, 0)),
        out_shape=jax.ShapeDtypeStruct((N_PAD, OUT_PAD), jnp.float32),
    )(pa, pb, b2p)


@jax.jit
def _run(edge_index, edge_weight, x, w1, b1, w2, b2):
    src = edge_index[0].astype(jnp.int32).reshape(NW, NG, GE)
    dst = edge_index[1].astype(jnp.int32).reshape(NW, NG, GE)
    w3 = edge_weight.reshape(NW, NG, GE)

    z_hid = jnp.zeros((N_PAD, HID), jnp.float32)
    z_out = jnp.zeros((N_PAD, OUT_PAD), jnp.float32)
    w2p = jnp.pad(w2, ((0, 0), (0, OUT_PAD - OUT)))
    b2p = jnp.pad(b2, (0, OUT_PAD - OUT)).reshape(1, OUT_PAD)
    b1r = b1.reshape(1, HID)

    y1 = _mm1(x, w1)
    p1 = _sc_prop_hid(src, dst, w3, y1, z_hid)
    y2 = _mid(p1[0], p1[1], b1r, w2p)
    p2 = _sc_prop_out(src, dst, w3, y2[:N_NODES], z_out)
    out = _fin(p2[0], p2[1], b2p)
    return out[:N_NODES, :OUT]


def kernel(edge_index, edge_weight, x, W1, b1, W2, b2):
    return _run(edge_index, edge_weight, x, W1, b1, W2, b2)
